# Initial kernel scaffold; baseline (speedup 1.0000x reference)
import jax
import jax.numpy as jnp
from jax.experimental import pallas as pl


def kernel(nodes_matrix, edge_index, global_vector, edges_matrix, params):
    raise NotImplementedError("write your pallas kernel here")


# speedup vs baseline: 2.5517x; 2.5517x over previous
import functools

import jax
import jax.numpy as jnp
from jax import lax
from jax.experimental import pallas as pl
from jax.experimental.pallas import tpu as pltpu
from jax.experimental.pallas import tpu_sc as plsc

F32 = jnp.float32
NC = 2
NS = 16
NW = NC * NS
CH = 400
HF = 2560
HP = 2688
NQ = 4
DR = 128


def _mm_body(x_ref, w_ref, o_ref):
    o_ref[...] = jnp.dot(x_ref[...], w_ref[...], preferred_element_type=F32)


def _mm(x, w, br):
    r, k = x.shape
    d = w.shape[1]
    return pl.pallas_call(
        _mm_body,
        grid=(r // br,),
        in_specs=[pl.BlockSpec((br, k), lambda i: (i, 0)),
                  pl.BlockSpec((k, d), lambda i: (0, 0))],
        out_specs=pl.BlockSpec((br, d), lambda i: (i, 0)),
        out_shape=jax.ShapeDtypeStruct((r, d), F32),
    )(x, w)


def _eg_body(e_ref, we_ref, g_ref, wg_ref, o_ref):
    gg = jnp.dot(g_ref[...], wg_ref[...], preferred_element_type=F32)
    o_ref[...] = jnp.dot(e_ref[...], we_ref[...],
                         preferred_element_type=F32) + gg


def _eg(e, we, g, wg, br):
    r, k = e.shape
    d = we.shape[1]
    kg = wg.shape[0]
    return pl.pallas_call(
        _eg_body,
        grid=(r // br,),
        in_specs=[pl.BlockSpec((br, k), lambda i: (i, 0)),
                  pl.BlockSpec((k, d), lambda i: (0, 0)),
                  pl.BlockSpec((1, kg), lambda i: (0, 0)),
                  pl.BlockSpec((kg, d), lambda i: (0, 0))],
        out_specs=pl.BlockSpec((br, d), lambda i: (i, 0)),
        out_shape=jax.ShapeDtypeStruct((r, d), F32),
    )(e, we, g.reshape(1, -1), wg)


def _logits_body(mi_ref, mj_ref, eg_ref, att_ref, lg_ref, bm_ref):
    s = mi_ref[...] + mj_ref[...] + eg_ref[...]
    z = jnp.where(s >= 0, s, 0.2 * s)
    lg = jnp.dot(z, att_ref[...], preferred_element_type=F32)
    lg_ref[...] = lg
    bm_ref[0, 0, :] = jnp.full((128,), jnp.max(lg), F32)


def _logits(mi, mj, eg, att, br):
    e, d = mi.shape
    nb = e // br
    return pl.pallas_call(
        _logits_body,
        grid=(nb,),
        in_specs=[pl.BlockSpec((br, d), lambda i: (i, 0)),
                  pl.BlockSpec((br, d), lambda i: (i, 0)),
                  pl.BlockSpec((br, d), lambda i: (i, 0)),
                  pl.BlockSpec((d, 1), lambda i: (0, 0))],
        out_specs=[pl.BlockSpec((br, 1), lambda i: (i, 0)),
                   pl.BlockSpec((1, 1, 128), lambda i: (i, 0, 0))],
        out_shape=[jax.ShapeDtypeStruct((e, 1), F32),
                   jax.ShapeDtypeStruct((nb, 1, 128), F32)],
    )(mi, mj, eg, att.reshape(d, 1))


def _aux_body(lg_ref, bm_ref, dst_ref, vd_ref, iq_ref, id_ref):
    gmax = jnp.max(bm_ref[...])
    ex = jnp.exp(lg_ref[...] - gmax)
    dst = dst_ref[...]
    br = dst.shape[0]
    lane = lax.broadcasted_iota(jnp.int32, (br, 128), 1)
    vd_ref[...] = jnp.where(lane == dst % 128, ex, 0.0)
    id_ref[...] = dst // 128
    gq = lax.broadcasted_iota(jnp.int32, (NQ, br, 1), 0)
    loc = jnp.broadcast_to(dst, (NQ, br, 1)) - gq * HF
    ok = (loc >= 0) & (loc < HF)
    iq_ref[...] = jnp.where(ok, loc, HF)


def _aux(lg, bm, dst2, br):
    e = lg.shape[0]
    nb = e // br
    bs = bm.shape
    return pl.pallas_call(
        _aux_body,
        grid=(nb,),
        in_specs=[pl.BlockSpec((br, 1), lambda i: (i, 0)),
                  pl.BlockSpec(bs, lambda i: (0, 0, 0)),
                  pl.BlockSpec((br, 1), lambda i: (i, 0))],
        out_specs=[pl.BlockSpec((br, 128), lambda i: (i, 0)),
                   pl.BlockSpec((NQ, br, 1), lambda i: (0, i, 0)),
                   pl.BlockSpec((br, 1), lambda i: (i, 0))],
        out_shape=[jax.ShapeDtypeStruct((e, 128), F32),
                   jax.ShapeDtypeStruct((NQ, e, 1), jnp.int32),
                   jax.ShapeDtypeStruct((e, 1), jnp.int32)],
    )(lg, bm, dst2)


def _vals_big_body(lg_ref, bm_ref, mj_ref, v_ref):
    gmax = jnp.max(bm_ref[...])
    ex = jnp.exp(lg_ref[...] - gmax)
    v_ref[0] = ex * mj_ref[...]


def _vals_big(lg, bm, mj, br):
    e = lg.shape[0]
    nb = e // br
    bs = bm.shape
    return pl.pallas_call(
        _vals_big_body,
        grid=(nb, 2),
        in_specs=[pl.BlockSpec((br, 1), lambda i, c: (i, 0)),
                  pl.BlockSpec(bs, lambda i, c: (0, 0, 0)),
                  pl.BlockSpec((br, 128), lambda i, c: (i, c))],
        out_specs=pl.BlockSpec((1, br, 128), lambda i, c: (c, i, 0)),
        out_shape=jax.ShapeDtypeStruct((2, e, 128), F32),
    )(lg, bm, mj)


def _vals_small_body(lg_ref, bm_ref, mj_ref, v_ref):
    gmax = jnp.max(bm_ref[...])
    ex = jnp.exp(lg_ref[...] - gmax)
    v_ref[...] = ex * mj_ref[...]


def _vals_small(lg, bm, mj, br):
    e = lg.shape[0]
    nb = e // br
    bs = bm.shape
    return pl.pallas_call(
        _vals_small_body,
        grid=(nb,),
        in_specs=[pl.BlockSpec((br, 1), lambda i: (i, 0)),
                  pl.BlockSpec(bs, lambda i: (0, 0, 0)),
                  pl.BlockSpec((br, 128), lambda i: (i, 0))],
        out_specs=pl.BlockSpec((br, 128), lambda i: (i, 0)),
        out_shape=jax.ShapeDtypeStruct((e, 128), F32),
    )(lg, bm, mj)


def _comb_big_body(u_ref, den_ref, b_ref, o_ref):
    den = den_ref[...] + 1e-16
    u = jnp.concatenate([u_ref[0], u_ref[1]], axis=1)
    o_ref[...] = u / den + b_ref[...]


def _comb_big(u, den, b, n, br):
    return pl.pallas_call(
        _comb_big_body,
        grid=(n // br,),
        in_specs=[pl.BlockSpec((2, br, 128), lambda i: (0, i, 0)),
                  pl.BlockSpec((br, 1), lambda i: (i, 0)),
                  pl.BlockSpec((1, 256), lambda i: (0, 0))],
        out_specs=pl.BlockSpec((br, 256), lambda i: (i, 0)),
        out_shape=jax.ShapeDtypeStruct((n, 256), F32),
    )(u, den, b.reshape(1, -1))


def _comb_small_body(u_ref, den_ref, sel_ref, b_ref, o_ref):
    den = den_ref[...] + 1e-16
    u = jnp.dot(u_ref[...], sel_ref[...], preferred_element_type=F32)
    o_ref[...] = u / den + b_ref[...]


def _comb_small(u, den, b, n, br):
    sel = jnp.eye(128, 16, dtype=F32)
    return pl.pallas_call(
        _comb_small_body,
        grid=(n // br,),
        in_specs=[pl.BlockSpec((br, 128), lambda i: (i, 0)),
                  pl.BlockSpec((br, 1), lambda i: (i, 0)),
                  pl.BlockSpec((128, 16), lambda i: (0, 0)),
                  pl.BlockSpec((1, 16), lambda i: (0, 0))],
        out_specs=pl.BlockSpec((br, 16), lambda i: (i, 0)),
        out_shape=jax.ShapeDtypeStruct((n, 16), F32),
    )(u, den, sel, b.reshape(1, -1))


def _head_body(a_ref, v_ref, o_ref):
    a = a_ref[...]
    n = a.shape[0]
    col = lax.broadcasted_iota(jnp.int32, a.shape, 1)
    node_mask = col < 14
    sleep = jnp.sum(jnp.where(col == 15, a, 0.0)) / n
    mon = jnp.sum(jnp.where(col == 14, a, 0.0)) / n
    m = jnp.maximum(jnp.max(jnp.where(node_mask, a, -jnp.inf)),
                    jnp.maximum(sleep, mon))
    en = jnp.where(node_mask, jnp.exp(a - m), 0.0)
    es = jnp.exp(sleep - m)
    em = jnp.exp(mon - m)
    ssum = jnp.sum(en) + es + em
    e1 = jnp.sum(en * a) + es * sleep + em * mon
    lse = m + jnp.log(ssum)
    value = jnp.sum(jnp.where(col == 0, v_ref[...], 0.0))
    oc = lax.broadcasted_iota(jnp.int32, (1, 8), 1)
    out = jnp.where(oc == 0, sleep, 0.0)
    out = out + jnp.where(oc == 1, mon, 0.0)
    out = out + jnp.where(oc == 2, m - lse, 0.0)
    out = out + jnp.where(oc == 3, lse - e1 / ssum, 0.0)
    out = out + jnp.where(oc == 4, value, 0.0)
    o_ref[...] = out


def _head(a, v):
    n = a.shape[0]
    return pl.pallas_call(
        _head_body,
        in_specs=[pl.BlockSpec((n, 16), lambda: (0, 0)),
                  pl.BlockSpec((n, 16), lambda: (0, 0))],
        out_specs=pl.BlockSpec((1, 8), lambda: (0, 0)),
        out_shape=jax.ShapeDtypeStruct((1, 8), F32),
    )(a, v)


@functools.lru_cache(maxsize=None)
def _make_gather(n, e, d):
    per_tile = e // NW
    nchunks = per_tile // CH
    mesh = plsc.VectorSubcoreMesh(core_axis_name="c", subcore_axis_name="s")

    @functools.partial(
        pl.kernel, mesh=mesh,
        out_type=[jax.ShapeDtypeStruct((e, d), F32),
                  jax.ShapeDtypeStruct((e, d), F32)],
        scratch_types=[pltpu.VMEM((CH,), jnp.int32),
                       pltpu.VMEM((CH, d), F32),
                       pltpu.SemaphoreType.DMA],
    )
    def k(xl_hbm, xr_hbm, dst_hbm, src_hbm, mi_hbm, mj_hbm,
          idx_v, rows_v, sem):
        wid = lax.axis_index("s") * NC + lax.axis_index("c")
        base0 = wid * per_tile

        def body(i, carry):
            b = base0 + i * CH
            pltpu.sync_copy(dst_hbm.at[pl.ds(b, CH)], idx_v)
            pltpu.async_copy(xl_hbm.at[idx_v], rows_v, sem).wait()
            pltpu.sync_copy(rows_v, mi_hbm.at[pl.ds(b, CH)])
            pltpu.sync_copy(src_hbm.at[pl.ds(b, CH)], idx_v)
            pltpu.async_copy(xr_hbm.at[idx_v], rows_v, sem).wait()
            pltpu.sync_copy(rows_v, mj_hbm.at[pl.ds(b, CH)])
            return carry

        lax.fori_loop(0, nchunks, body, 0)

    return k


@functools.lru_cache(maxsize=None)
def _make_scatter(n, e, nchan):
    per_sub = (e if nchan == 2 else e // NC) // NS
    nch_v = per_sub // CH
    half = e // NC
    per_sub_d = half // NS
    nch_d = per_sub_d // CH
    stripe = HP // NS
    fl = HF // NS
    sd = DR // NS
    mesh = plsc.VectorSubcoreMesh(core_axis_name="c", subcore_axis_name="s")

    @functools.partial(
        pl.kernel, mesh=mesh,
        out_type=[jax.ShapeDtypeStruct((NC * NQ * HF, 128), F32),
                  jax.ShapeDtypeStruct((NC * DR, 128), F32)],
        scratch_types=[pltpu.VMEM((CH,), jnp.int32),
                       pltpu.VMEM((CH, 128), F32),
                       pltpu.VMEM_SHARED((HP, 128), F32),
                       pltpu.VMEM_SHARED((DR, 128), F32)],
    )
    def k(vals_hbm, vd_hbm, iq_hbm, id_hbm, zu_hbm, u_hbm, d_hbm,
          idx_v, vv, sh_u, sh_d):
        c = lax.axis_index("c")
        s = lax.axis_index("s")
        for g in range(NQ):
            r0 = s * stripe
            pltpu.sync_copy(zu_hbm.at[pl.ds(r0, stripe)],
                            sh_u.at[pl.ds(r0, stripe)])
            plsc.subcore_barrier()

            def body(i, carry):
                if nchan == 2:
                    be = s * per_sub + i * CH
                    bv = c * e + be
                else:
                    be = c * half + s * per_sub + i * CH
                    bv = be
                pltpu.sync_copy(iq_hbm.at[pl.ds(g * e + be, CH)], idx_v)
                pltpu.sync_copy(vals_hbm.at[pl.ds(bv, CH)], vv)
                pltpu.sync_copy(vv, sh_u.at[idx_v], add=True)
                return carry

            lax.fori_loop(0, nch_v, body, 0)
            plsc.subcore_barrier()
            pltpu.sync_copy(
                sh_u.at[pl.ds(s * fl, fl)],
                u_hbm.at[pl.ds(c * (NQ * HF) + g * HF + s * fl, fl)])
            plsc.subcore_barrier()
        rd = s * sd
        pltpu.sync_copy(zu_hbm.at[pl.ds(rd, sd)], sh_d.at[pl.ds(rd, sd)])
        plsc.subcore_barrier()

        def body_d(i, carry):
            b = c * half + s * per_sub_d + i * CH
            pltpu.sync_copy(id_hbm.at[pl.ds(b, CH)], idx_v)
            pltpu.sync_copy(vd_hbm.at[pl.ds(b, CH)], vv)
            pltpu.sync_copy(vv, sh_d.at[idx_v], add=True)
            return carry

        lax.fori_loop(0, nch_d, body_d, 0)
        plsc.subcore_barrier()
        pltpu.sync_copy(sh_d.at[pl.ds(rd, sd)],
                        d_hbm.at[pl.ds(c * DR + rd, sd)])

    return k


def _gat_layer(x, src, dst, dst2, e_feat, g, p, big):
    n = x.shape[0]
    e = e_feat.shape[0]
    br_n = 1000
    br_e = 1000
    if big:
        d = p['Wl'].shape[1]
        wl, wr, we, wg, att = p['Wl'], p['Wr'], p['We'], p['Wg'], p['att']
        b = p['b']
    else:
        d = 128

        def pad_w(w):
            return jnp.pad(w, ((0, 0), (0, d - w.shape[1])))

        wl, wr, we, wg = map(pad_w, (p['Wl'], p['Wr'], p['We'], p['Wg']))
        att = jnp.pad(p['att'], (0, d - p['att'].shape[0]))
        b = jnp.pad(p['b'], (0, 16 - p['b'].shape[0]))
    xl = _mm(x, wl, br_n)
    xr = _mm(x, wr, br_n)
    eg = _eg(e_feat, we, g, wg, br_e)
    mi, mj = _make_gather(n, e, d)(xl, xr, dst, src)
    lg, bm = _logits(mi, mj, eg, att, br_e)
    vd, iq, idd = _aux(lg, bm, dst2, br_e)
    iqf = iq.reshape(NQ * e)
    idf = idd.reshape(e)
    zu = jnp.zeros((HP, 128), F32)
    if big:
        vals = _vals_big(lg, bm, mj, br_e).reshape(2 * e, 128)
        u, dd = _make_scatter(n, e, 2)(vals, vd, iqf, idf, zu)
    else:
        vals = _vals_small(lg, bm, mj, br_e)
        u, dd = _make_scatter(n, e, 1)(vals, vd, iqf, idf, zu)
    u = u.reshape(NC, NQ * HF, 128)
    dd = dd.reshape(NC, DR, 128)
    den = (dd[0] + dd[1]).reshape(DR * 128)[:n].reshape(n, 1)
    if big:
        return _comb_big(u, den, b, n, br_n)
    return _comb_small(u[0] + u[1], den, b, n, br_n)


def kernel(nodes_matrix, edge_index, global_vector, edges_matrix, params):
    src = edge_index[0]
    dst = edge_index[1]
    dst2 = dst.reshape(-1, 1)
    g = global_vector
    ef = edges_matrix
    h = _gat_layer(nodes_matrix, src, dst, dst2, ef, g, params['a0'], True)
    h = _gat_layer(h, src, dst, dst2, ef, g, params['a1'], True)
    a = _gat_layer(h, src, dst, dst2, ef, g, params['ah'], False)
    c = _gat_layer(nodes_matrix, src, dst, dst2, ef, g, params['c0'], True)
    c = _gat_layer(c, src, dst, dst2, ef, g, params['c1'], True)
    v = _gat_layer(c, src, dst, dst2, ef, g, params['ch'], False)
    o = _head(a, v)
    flat = jnp.concatenate([o[0, 0:1], o[0, 1:2], a[:, :14].reshape(-1)])
    return (flat, o[0, 2], o[0, 3], o[0, 4])


# speedup vs baseline: 3.0246x; 1.1853x over previous
import functools

import jax
import jax.numpy as jnp
from jax import lax
from jax.experimental import pallas as pl
from jax.experimental.pallas import tpu as pltpu
from jax.experimental.pallas import tpu_sc as plsc

F32 = jnp.float32
NC = 2
NS = 16
NW = NC * NS
CH = 400
HF = 5120
HP = 5248
NQ = 2
DR = 128


def _mm_body(x_ref, w_ref, o_ref):
    o_ref[...] = jnp.dot(x_ref[...], w_ref[...], preferred_element_type=F32)


def _mm(x, w, br):
    r, k = x.shape
    d = w.shape[1]
    return pl.pallas_call(
        _mm_body,
        grid=(r // br,),
        in_specs=[pl.BlockSpec((br, k), lambda i: (i, 0)),
                  pl.BlockSpec((k, d), lambda i: (0, 0))],
        out_specs=pl.BlockSpec((br, d), lambda i: (i, 0)),
        out_shape=jax.ShapeDtypeStruct((r, d), F32),
    )(x, w)


def _eg_body(e_ref, we_ref, g_ref, wg_ref, o_ref):
    gg = jnp.dot(g_ref[...], wg_ref[...], preferred_element_type=F32)
    o_ref[...] = jnp.dot(e_ref[...], we_ref[...],
                         preferred_element_type=F32) + gg


def _eg(e, we, g, wg, br):
    r, k = e.shape
    d = we.shape[1]
    kg = wg.shape[0]
    return pl.pallas_call(
        _eg_body,
        grid=(r // br,),
        in_specs=[pl.BlockSpec((br, k), lambda i: (i, 0)),
                  pl.BlockSpec((k, d), lambda i: (0, 0)),
                  pl.BlockSpec((1, kg), lambda i: (0, 0)),
                  pl.BlockSpec((kg, d), lambda i: (0, 0))],
        out_specs=pl.BlockSpec((br, d), lambda i: (i, 0)),
        out_shape=jax.ShapeDtypeStruct((r, d), F32),
    )(e, we, g.reshape(1, -1), wg)


def _logits_body(mi_ref, mj_ref, eg_ref, att_ref, lg_ref, bm_ref):
    s = mi_ref[...] + mj_ref[...] + eg_ref[...]
    z = jnp.where(s >= 0, s, 0.2 * s)
    lg = jnp.dot(z, att_ref[...], preferred_element_type=F32)
    lg_ref[...] = lg
    bm_ref[0, 0, :] = jnp.full((128,), jnp.max(lg), F32)


def _logits(mi, mj, eg, att, br):
    e, d = mi.shape
    nb = e // br
    return pl.pallas_call(
        _logits_body,
        grid=(nb,),
        in_specs=[pl.BlockSpec((br, d), lambda i: (i, 0)),
                  pl.BlockSpec((br, d), lambda i: (i, 0)),
                  pl.BlockSpec((br, d), lambda i: (i, 0)),
                  pl.BlockSpec((d, 1), lambda i: (0, 0))],
        out_specs=[pl.BlockSpec((br, 1), lambda i: (i, 0)),
                   pl.BlockSpec((1, 1, 128), lambda i: (i, 0, 0))],
        out_shape=[jax.ShapeDtypeStruct((e, 1), F32),
                   jax.ShapeDtypeStruct((nb, 1, 128), F32)],
    )(mi, mj, eg, att.reshape(d, 1))


def _aux_body(lg_ref, bm_ref, dst_ref, vd_ref, iq_ref, id_ref):
    gmax = jnp.max(bm_ref[...])
    ex = jnp.exp(lg_ref[...] - gmax)
    dst = dst_ref[...]
    br = dst.shape[0]
    lane = lax.broadcasted_iota(jnp.int32, (br, 128), 1)
    vd_ref[...] = jnp.where(lane == dst % 128, ex, 0.0)
    id_ref[...] = dst // 128
    gq = lax.broadcasted_iota(jnp.int32, (NQ, br, 1), 0)
    loc = jnp.broadcast_to(dst, (NQ, br, 1)) - gq * HF
    ok = (loc >= 0) & (loc < HF)
    iq_ref[...] = jnp.where(ok, loc, HF)


def _aux(lg, bm, dst2, br):
    e = lg.shape[0]
    nb = e // br
    bs = bm.shape
    return pl.pallas_call(
        _aux_body,
        grid=(nb,),
        in_specs=[pl.BlockSpec((br, 1), lambda i: (i, 0)),
                  pl.BlockSpec(bs, lambda i: (0, 0, 0)),
                  pl.BlockSpec((br, 1), lambda i: (i, 0))],
        out_specs=[pl.BlockSpec((br, 128), lambda i: (i, 0)),
                   pl.BlockSpec((NQ, br, 1), lambda i: (0, i, 0)),
                   pl.BlockSpec((br, 1), lambda i: (i, 0))],
        out_shape=[jax.ShapeDtypeStruct((e, 128), F32),
                   jax.ShapeDtypeStruct((NQ, e, 1), jnp.int32),
                   jax.ShapeDtypeStruct((e, 1), jnp.int32)],
    )(lg, bm, dst2)


def _vals_big_body(lg_ref, bm_ref, mj_ref, v_ref):
    gmax = jnp.max(bm_ref[...])
    ex = jnp.exp(lg_ref[...] - gmax)
    v_ref[0] = ex * mj_ref[...]


def _vals_big(lg, bm, mj, br):
    e = lg.shape[0]
    nb = e // br
    bs = bm.shape
    return pl.pallas_call(
        _vals_big_body,
        grid=(nb, 2),
        in_specs=[pl.BlockSpec((br, 1), lambda i, c: (i, 0)),
                  pl.BlockSpec(bs, lambda i, c: (0, 0, 0)),
                  pl.BlockSpec((br, 128), lambda i, c: (i, c))],
        out_specs=pl.BlockSpec((1, br, 128), lambda i, c: (c, i, 0)),
        out_shape=jax.ShapeDtypeStruct((2, e, 128), F32),
    )(lg, bm, mj)


def _vals_small_body(lg_ref, bm_ref, mj_ref, v_ref):
    gmax = jnp.max(bm_ref[...])
    ex = jnp.exp(lg_ref[...] - gmax)
    v_ref[...] = ex * mj_ref[...]


def _vals_small(lg, bm, mj, br):
    e = lg.shape[0]
    nb = e // br
    bs = bm.shape
    return pl.pallas_call(
        _vals_small_body,
        grid=(nb,),
        in_specs=[pl.BlockSpec((br, 1), lambda i: (i, 0)),
                  pl.BlockSpec(bs, lambda i: (0, 0, 0)),
                  pl.BlockSpec((br, 128), lambda i: (i, 0))],
        out_specs=pl.BlockSpec((br, 128), lambda i: (i, 0)),
        out_shape=jax.ShapeDtypeStruct((e, 128), F32),
    )(lg, bm, mj)


def _comb_big_body(u_ref, den_ref, b_ref, o_ref):
    den = den_ref[...] + 1e-16
    u = jnp.concatenate([u_ref[0], u_ref[1]], axis=1)
    o_ref[...] = u / den + b_ref[...]


def _comb_big(u, den, b, n, br):
    return pl.pallas_call(
        _comb_big_body,
        grid=(n // br,),
        in_specs=[pl.BlockSpec((2, br, 128), lambda i: (0, i, 0)),
                  pl.BlockSpec((br, 1), lambda i: (i, 0)),
                  pl.BlockSpec((1, 256), lambda i: (0, 0))],
        out_specs=pl.BlockSpec((br, 256), lambda i: (i, 0)),
        out_shape=jax.ShapeDtypeStruct((n, 256), F32),
    )(u, den, b.reshape(1, -1))


def _comb_small_body(u_ref, den_ref, sel_ref, b_ref, o_ref):
    den = den_ref[...] + 1e-16
    u = jnp.dot(u_ref[...], sel_ref[...], preferred_element_type=F32)
    o_ref[...] = u / den + b_ref[...]


def _comb_small(u, den, b, n, br):
    sel = jnp.eye(128, 16, dtype=F32)
    return pl.pallas_call(
        _comb_small_body,
        grid=(n // br,),
        in_specs=[pl.BlockSpec((br, 128), lambda i: (i, 0)),
                  pl.BlockSpec((br, 1), lambda i: (i, 0)),
                  pl.BlockSpec((128, 16), lambda i: (0, 0)),
                  pl.BlockSpec((1, 16), lambda i: (0, 0))],
        out_specs=pl.BlockSpec((br, 16), lambda i: (i, 0)),
        out_shape=jax.ShapeDtypeStruct((n, 16), F32),
    )(u, den, sel, b.reshape(1, -1))


def _head_body(a_ref, v_ref, o_ref):
    a = a_ref[...]
    n = a.shape[0]
    col = lax.broadcasted_iota(jnp.int32, a.shape, 1)
    node_mask = col < 14
    sleep = jnp.sum(jnp.where(col == 15, a, 0.0)) / n
    mon = jnp.sum(jnp.where(col == 14, a, 0.0)) / n
    m = jnp.maximum(jnp.max(jnp.where(node_mask, a, -jnp.inf)),
                    jnp.maximum(sleep, mon))
    en = jnp.where(node_mask, jnp.exp(a - m), 0.0)
    es = jnp.exp(sleep - m)
    em = jnp.exp(mon - m)
    ssum = jnp.sum(en) + es + em
    e1 = jnp.sum(en * a) + es * sleep + em * mon
    lse = m + jnp.log(ssum)
    value = jnp.sum(jnp.where(col == 0, v_ref[...], 0.0))
    oc = lax.broadcasted_iota(jnp.int32, (1, 8), 1)
    out = jnp.where(oc == 0, sleep, 0.0)
    out = out + jnp.where(oc == 1, mon, 0.0)
    out = out + jnp.where(oc == 2, m - lse, 0.0)
    out = out + jnp.where(oc == 3, lse - e1 / ssum, 0.0)
    out = out + jnp.where(oc == 4, value, 0.0)
    o_ref[...] = out


def _head(a, v):
    n = a.shape[0]
    return pl.pallas_call(
        _head_body,
        in_specs=[pl.BlockSpec((n, 16), lambda: (0, 0)),
                  pl.BlockSpec((n, 16), lambda: (0, 0))],
        out_specs=pl.BlockSpec((1, 8), lambda: (0, 0)),
        out_shape=jax.ShapeDtypeStruct((1, 8), F32),
    )(a, v)


@functools.lru_cache(maxsize=None)
def _make_gather(n, e, d):
    per_tile = e // NW
    nchunks = per_tile // CH
    mesh = plsc.VectorSubcoreMesh(core_axis_name="c", subcore_axis_name="s")

    @functools.partial(
        pl.kernel, mesh=mesh,
        out_type=[jax.ShapeDtypeStruct((e, d), F32),
                  jax.ShapeDtypeStruct((e, d), F32)],
        scratch_types=[pltpu.VMEM((CH,), jnp.int32),
                       pltpu.VMEM((CH, d), F32),
                       pltpu.SemaphoreType.DMA],
    )
    def k(xl_hbm, xr_hbm, dst_hbm, src_hbm, mi_hbm, mj_hbm,
          idx_v, rows_v, sem):
        wid = lax.axis_index("s") * NC + lax.axis_index("c")
        base0 = wid * per_tile

        def body(i, carry):
            b = base0 + i * CH
            pltpu.sync_copy(dst_hbm.at[pl.ds(b, CH)], idx_v)
            pltpu.async_copy(xl_hbm.at[idx_v], rows_v, sem).wait()
            pltpu.sync_copy(rows_v, mi_hbm.at[pl.ds(b, CH)])
            pltpu.sync_copy(src_hbm.at[pl.ds(b, CH)], idx_v)
            pltpu.async_copy(xr_hbm.at[idx_v], rows_v, sem).wait()
            pltpu.sync_copy(rows_v, mj_hbm.at[pl.ds(b, CH)])
            return carry

        lax.fori_loop(0, nchunks, body, 0)

    return k


@functools.lru_cache(maxsize=None)
def _make_scatter(n, e, nchan):
    per_sub = (e if nchan == 2 else e // NC) // NS
    nch_v = per_sub // CH
    half = e // NC
    per_sub_d = half // NS
    nch_d = per_sub_d // CH
    stripe = HP // NS
    fl = HF // NS
    sd = DR // NS
    mesh = plsc.VectorSubcoreMesh(core_axis_name="c", subcore_axis_name="s")

    @functools.partial(
        pl.kernel, mesh=mesh,
        out_type=[jax.ShapeDtypeStruct((NC * NQ * HF, 128), F32),
                  jax.ShapeDtypeStruct((NC * DR, 128), F32)],
        scratch_types=[pltpu.VMEM((CH,), jnp.int32),
                       pltpu.VMEM((CH, 128), F32),
                       pltpu.VMEM_SHARED((HP, 128), F32),
                       pltpu.VMEM_SHARED((DR, 128), F32)],
    )
    def k(vals_hbm, vd_hbm, iq_hbm, id_hbm, zu_hbm, u_hbm, d_hbm,
          idx_v, vv, sh_u, sh_d):
        c = lax.axis_index("c")
        s = lax.axis_index("s")
        for g in range(NQ):
            r0 = s * stripe
            pltpu.sync_copy(zu_hbm.at[pl.ds(r0, stripe)],
                            sh_u.at[pl.ds(r0, stripe)])
            plsc.subcore_barrier()

            def body(i, carry):
                if nchan == 2:
                    be = s * per_sub + i * CH
                    bv = c * e + be
                else:
                    be = c * half + s * per_sub + i * CH
                    bv = be
                pltpu.sync_copy(iq_hbm.at[pl.ds(g * e + be, CH)], idx_v)
                pltpu.sync_copy(vals_hbm.at[pl.ds(bv, CH)], vv)
                pltpu.sync_copy(vv, sh_u.at[idx_v], add=True)
                return carry

            lax.fori_loop(0, nch_v, body, 0)
            plsc.subcore_barrier()
            pltpu.sync_copy(
                sh_u.at[pl.ds(s * fl, fl)],
                u_hbm.at[pl.ds(c * (NQ * HF) + g * HF + s * fl, fl)])
            plsc.subcore_barrier()
        rd = s * sd
        pltpu.sync_copy(zu_hbm.at[pl.ds(rd, sd)], sh_d.at[pl.ds(rd, sd)])
        plsc.subcore_barrier()

        def body_d(i, carry):
            b = c * half + s * per_sub_d + i * CH
            pltpu.sync_copy(id_hbm.at[pl.ds(b, CH)], idx_v)
            pltpu.sync_copy(vd_hbm.at[pl.ds(b, CH)], vv)
            pltpu.sync_copy(vv, sh_d.at[idx_v], add=True)
            return carry

        lax.fori_loop(0, nch_d, body_d, 0)
        plsc.subcore_barrier()
        pltpu.sync_copy(sh_d.at[pl.ds(rd, sd)],
                        d_hbm.at[pl.ds(c * DR + rd, sd)])

    return k


def _gat_layer(x, src, dst, dst2, e_feat, g, p, big):
    n = x.shape[0]
    e = e_feat.shape[0]
    br_n = 1000
    br_e = 1000
    if big:
        d = p['Wl'].shape[1]
        wl, wr, we, wg, att = p['Wl'], p['Wr'], p['We'], p['Wg'], p['att']
        b = p['b']
    else:
        d = 128

        def pad_w(w):
            return jnp.pad(w, ((0, 0), (0, d - w.shape[1])))

        wl, wr, we, wg = map(pad_w, (p['Wl'], p['Wr'], p['We'], p['Wg']))
        att = jnp.pad(p['att'], (0, d - p['att'].shape[0]))
        b = jnp.pad(p['b'], (0, 16 - p['b'].shape[0]))
    xl = _mm(x, wl, br_n)
    xr = _mm(x, wr, br_n)
    eg = _eg(e_feat, we, g, wg, br_e)
    mi, mj = _make_gather(n, e, d)(xl, xr, dst, src)
    lg, bm = _logits(mi, mj, eg, att, br_e)
    vd, iq, idd = _aux(lg, bm, dst2, br_e)
    iqf = iq.reshape(NQ * e)
    idf = idd.reshape(e)
    zu = jnp.zeros((HP, 128), F32)
    if big:
        vals = _vals_big(lg, bm, mj, br_e).reshape(2 * e, 128)
        u, dd = _make_scatter(n, e, 2)(vals, vd, iqf, idf, zu)
    else:
        vals = _vals_small(lg, bm, mj, br_e)
        u, dd = _make_scatter(n, e, 1)(vals, vd, iqf, idf, zu)
    u = u.reshape(NC, NQ * HF, 128)
    dd = dd.reshape(NC, DR, 128)
    den = (dd[0] + dd[1]).reshape(DR * 128)[:n].reshape(n, 1)
    if big:
        return _comb_big(u, den, b, n, br_n)
    return _comb_small(u[0] + u[1], den, b, n, br_n)


def kernel(nodes_matrix, edge_index, global_vector, edges_matrix, params):
    src = edge_index[0]
    dst = edge_index[1]
    dst2 = dst.reshape(-1, 1)
    g = global_vector
    ef = edges_matrix
    h = _gat_layer(nodes_matrix, src, dst, dst2, ef, g, params['a0'], True)
    h = _gat_layer(h, src, dst, dst2, ef, g, params['a1'], True)
    a = _gat_layer(h, src, dst, dst2, ef, g, params['ah'], False)
    c = _gat_layer(nodes_matrix, src, dst, dst2, ef, g, params['c0'], True)
    c = _gat_layer(c, src, dst, dst2, ef, g, params['c1'], True)
    v = _gat_layer(c, src, dst, dst2, ef, g, params['ch'], False)
    o = _head(a, v)
    flat = jnp.concatenate([o[0, 0:1], o[0, 1:2], a[:, :14].reshape(-1)])
    return (flat, o[0, 2], o[0, 3], o[0, 4])


# speedup vs baseline: 3.0289x; 1.0014x over previous
import functools

import jax
import jax.numpy as jnp
from jax import lax
from jax.experimental import pallas as pl
from jax.experimental.pallas import tpu as pltpu
from jax.experimental.pallas import tpu_sc as plsc

F32 = jnp.float32
NC = 2
NS = 16
NW = NC * NS
CH = 400
HF = 5120
HP = 5248
NQ = 2
DR = 128


def _mm_body(x_ref, w_ref, o_ref):
    o_ref[...] = jnp.dot(x_ref[...], w_ref[...], preferred_element_type=F32)


def _mm(x, w, br):
    r, k = x.shape
    d = w.shape[1]
    return pl.pallas_call(
        _mm_body,
        grid=(r // br,),
        in_specs=[pl.BlockSpec((br, k), lambda i: (i, 0)),
                  pl.BlockSpec((k, d), lambda i: (0, 0))],
        out_specs=pl.BlockSpec((br, d), lambda i: (i, 0)),
        out_shape=jax.ShapeDtypeStruct((r, d), F32),
    )(x, w)


def _eg_body(e_ref, we_ref, g_ref, wg_ref, o_ref):
    gg = jnp.dot(g_ref[...], wg_ref[...], preferred_element_type=F32)
    o_ref[...] = jnp.dot(e_ref[...], we_ref[...],
                         preferred_element_type=F32) + gg


def _eg(e, we, g, wg, br):
    r, k = e.shape
    d = we.shape[1]
    kg = wg.shape[0]
    return pl.pallas_call(
        _eg_body,
        grid=(r // br,),
        in_specs=[pl.BlockSpec((br, k), lambda i: (i, 0)),
                  pl.BlockSpec((k, d), lambda i: (0, 0)),
                  pl.BlockSpec((1, kg), lambda i: (0, 0)),
                  pl.BlockSpec((kg, d), lambda i: (0, 0))],
        out_specs=pl.BlockSpec((br, d), lambda i: (i, 0)),
        out_shape=jax.ShapeDtypeStruct((r, d), F32),
    )(e, we, g.reshape(1, -1), wg)


def _logits_body(mi_ref, mj_ref, eg_ref, att_ref, lg_ref, bm_ref):
    s = mi_ref[...] + mj_ref[...] + eg_ref[...]
    z = jnp.where(s >= 0, s, 0.2 * s)
    lg = jnp.dot(z, att_ref[...], preferred_element_type=F32)
    lg_ref[...] = lg
    bm_ref[0, 0, :] = jnp.full((128,), jnp.max(lg), F32)


def _logits(mi, mj, eg, att, br):
    e, d = mi.shape
    nb = e // br
    return pl.pallas_call(
        _logits_body,
        grid=(nb,),
        in_specs=[pl.BlockSpec((br, d), lambda i: (i, 0)),
                  pl.BlockSpec((br, d), lambda i: (i, 0)),
                  pl.BlockSpec((br, d), lambda i: (i, 0)),
                  pl.BlockSpec((d, 1), lambda i: (0, 0))],
        out_specs=[pl.BlockSpec((br, 1), lambda i: (i, 0)),
                   pl.BlockSpec((1, 1, 128), lambda i: (i, 0, 0))],
        out_shape=[jax.ShapeDtypeStruct((e, 1), F32),
                   jax.ShapeDtypeStruct((nb, 1, 128), F32)],
    )(mi, mj, eg, att.reshape(d, 1))


def _aux_body(lg_ref, bm_ref, dst_ref, vd_ref, iq_ref, id_ref):
    gmax = jnp.max(bm_ref[...])
    ex = jnp.exp(lg_ref[...] - gmax)
    dst = dst_ref[...]
    br = dst.shape[0]
    lane = lax.broadcasted_iota(jnp.int32, (br, 128), 1)
    vd_ref[...] = jnp.where(lane == dst % 128, ex, 0.0)
    id_ref[...] = dst // 128
    gq = lax.broadcasted_iota(jnp.int32, (NQ, br, 1), 0)
    loc = jnp.broadcast_to(dst, (NQ, br, 1)) - gq * HF
    ok = (loc >= 0) & (loc < HF)
    iq_ref[...] = jnp.where(ok, loc, HF)


def _aux(lg, bm, dst2, br):
    e = lg.shape[0]
    nb = e // br
    bs = bm.shape
    return pl.pallas_call(
        _aux_body,
        grid=(nb,),
        in_specs=[pl.BlockSpec((br, 1), lambda i: (i, 0)),
                  pl.BlockSpec(bs, lambda i: (0, 0, 0)),
                  pl.BlockSpec((br, 1), lambda i: (i, 0))],
        out_specs=[pl.BlockSpec((br, 128), lambda i: (i, 0)),
                   pl.BlockSpec((NQ, br, 1), lambda i: (0, i, 0)),
                   pl.BlockSpec((br, 1), lambda i: (i, 0))],
        out_shape=[jax.ShapeDtypeStruct((e, 128), F32),
                   jax.ShapeDtypeStruct((NQ, e, 1), jnp.int32),
                   jax.ShapeDtypeStruct((e, 1), jnp.int32)],
    )(lg, bm, dst2)


def _vals_big_body(lg_ref, bm_ref, mj_ref, v_ref):
    gmax = jnp.max(bm_ref[...])
    ex = jnp.exp(lg_ref[...] - gmax)
    v_ref[0] = ex * mj_ref[...]


def _vals_big(lg, bm, mj, br):
    e = lg.shape[0]
    nb = e // br
    bs = bm.shape
    return pl.pallas_call(
        _vals_big_body,
        grid=(nb, 2),
        in_specs=[pl.BlockSpec((br, 1), lambda i, c: (i, 0)),
                  pl.BlockSpec(bs, lambda i, c: (0, 0, 0)),
                  pl.BlockSpec((br, 128), lambda i, c: (i, c))],
        out_specs=pl.BlockSpec((1, br, 128), lambda i, c: (c, i, 0)),
        out_shape=jax.ShapeDtypeStruct((2, e, 128), F32),
    )(lg, bm, mj)


def _vals_small_body(lg_ref, bm_ref, mj_ref, v_ref):
    gmax = jnp.max(bm_ref[...])
    ex = jnp.exp(lg_ref[...] - gmax)
    v_ref[...] = ex * mj_ref[...]


def _vals_small(lg, bm, mj, br):
    e = lg.shape[0]
    nb = e // br
    bs = bm.shape
    return pl.pallas_call(
        _vals_small_body,
        grid=(nb,),
        in_specs=[pl.BlockSpec((br, 1), lambda i: (i, 0)),
                  pl.BlockSpec(bs, lambda i: (0, 0, 0)),
                  pl.BlockSpec((br, 128), lambda i: (i, 0))],
        out_specs=pl.BlockSpec((br, 128), lambda i: (i, 0)),
        out_shape=jax.ShapeDtypeStruct((e, 128), F32),
    )(lg, bm, mj)


def _comb_big_body(u_ref, den_ref, b_ref, o_ref):
    den = den_ref[...] + 1e-16
    u = jnp.concatenate([u_ref[0], u_ref[1]], axis=1)
    o_ref[...] = u / den + b_ref[...]


def _comb_big(u, den, b, n, br):
    return pl.pallas_call(
        _comb_big_body,
        grid=(n // br,),
        in_specs=[pl.BlockSpec((2, br, 128), lambda i: (0, i, 0)),
                  pl.BlockSpec((br, 1), lambda i: (i, 0)),
                  pl.BlockSpec((1, 256), lambda i: (0, 0))],
        out_specs=pl.BlockSpec((br, 256), lambda i: (i, 0)),
        out_shape=jax.ShapeDtypeStruct((n, 256), F32),
    )(u, den, b.reshape(1, -1))


def _comb_small_body(u_ref, den_ref, sel_ref, b_ref, o_ref):
    den = den_ref[...] + 1e-16
    u = jnp.dot(u_ref[...], sel_ref[...], preferred_element_type=F32)
    o_ref[...] = u / den + b_ref[...]


def _comb_small(u, den, b, n, br):
    sel = jnp.eye(128, 16, dtype=F32)
    return pl.pallas_call(
        _comb_small_body,
        grid=(n // br,),
        in_specs=[pl.BlockSpec((br, 128), lambda i: (i, 0)),
                  pl.BlockSpec((br, 1), lambda i: (i, 0)),
                  pl.BlockSpec((128, 16), lambda i: (0, 0)),
                  pl.BlockSpec((1, 16), lambda i: (0, 0))],
        out_specs=pl.BlockSpec((br, 16), lambda i: (i, 0)),
        out_shape=jax.ShapeDtypeStruct((n, 16), F32),
    )(u, den, sel, b.reshape(1, -1))


def _head_body(a_ref, v_ref, o_ref):
    a = a_ref[...]
    n = a.shape[0]
    col = lax.broadcasted_iota(jnp.int32, a.shape, 1)
    node_mask = col < 14
    sleep = jnp.sum(jnp.where(col == 15, a, 0.0)) / n
    mon = jnp.sum(jnp.where(col == 14, a, 0.0)) / n
    m = jnp.maximum(jnp.max(jnp.where(node_mask, a, -jnp.inf)),
                    jnp.maximum(sleep, mon))
    en = jnp.where(node_mask, jnp.exp(a - m), 0.0)
    es = jnp.exp(sleep - m)
    em = jnp.exp(mon - m)
    ssum = jnp.sum(en) + es + em
    e1 = jnp.sum(en * a) + es * sleep + em * mon
    lse = m + jnp.log(ssum)
    value = jnp.sum(jnp.where(col == 0, v_ref[...], 0.0))
    oc = lax.broadcasted_iota(jnp.int32, (1, 8), 1)
    out = jnp.where(oc == 0, sleep, 0.0)
    out = out + jnp.where(oc == 1, mon, 0.0)
    out = out + jnp.where(oc == 2, m - lse, 0.0)
    out = out + jnp.where(oc == 3, lse - e1 / ssum, 0.0)
    out = out + jnp.where(oc == 4, value, 0.0)
    o_ref[...] = out


def _head(a, v):
    n = a.shape[0]
    return pl.pallas_call(
        _head_body,
        in_specs=[pl.BlockSpec((n, 16), lambda: (0, 0)),
                  pl.BlockSpec((n, 16), lambda: (0, 0))],
        out_specs=pl.BlockSpec((1, 8), lambda: (0, 0)),
        out_shape=jax.ShapeDtypeStruct((1, 8), F32),
    )(a, v)


@functools.lru_cache(maxsize=None)
def _make_gather(n, e, d):
    per_tile = e // NW
    nchunks = per_tile // CH
    mesh = plsc.VectorSubcoreMesh(core_axis_name="c", subcore_axis_name="s")

    @functools.partial(
        pl.kernel, mesh=mesh,
        out_type=[jax.ShapeDtypeStruct((e, d), F32),
                  jax.ShapeDtypeStruct((e, d), F32)],
        scratch_types=[pltpu.VMEM((CH,), jnp.int32),
                       pltpu.VMEM((CH, d), F32),
                       pltpu.SemaphoreType.DMA],
    )
    def k(xl_hbm, xr_hbm, dst_hbm, src_hbm, mi_hbm, mj_hbm,
          idx_v, rows_v, sem):
        wid = lax.axis_index("s") * NC + lax.axis_index("c")
        base0 = wid * per_tile

        def body(i, carry):
            b = base0 + i * CH
            pltpu.sync_copy(dst_hbm.at[pl.ds(b, CH)], idx_v)
            pltpu.async_copy(xl_hbm.at[idx_v], rows_v, sem).wait()
            pltpu.sync_copy(rows_v, mi_hbm.at[pl.ds(b, CH)])
            pltpu.sync_copy(src_hbm.at[pl.ds(b, CH)], idx_v)
            pltpu.async_copy(xr_hbm.at[idx_v], rows_v, sem).wait()
            pltpu.sync_copy(rows_v, mj_hbm.at[pl.ds(b, CH)])
            return carry

        lax.fori_loop(0, nchunks, body, 0)

    return k


@functools.lru_cache(maxsize=None)
def _make_scatter(n, e, nchan):
    per_sub = (e if nchan == 2 else e // NC) // NS
    CHV = CH
    nch_v = per_sub // CHV
    half = e // NC
    per_sub_d = half // NS
    nch_d = per_sub_d // CH
    stripe = HP // NS
    fl = HF // NS
    sd = DR // NS
    mesh = plsc.VectorSubcoreMesh(core_axis_name="c", subcore_axis_name="s")

    @functools.partial(
        pl.kernel, mesh=mesh,
        out_type=[jax.ShapeDtypeStruct((NC * NQ * HF, 128), F32),
                  jax.ShapeDtypeStruct((NC * DR, 128), F32)],
        scratch_types=[pltpu.VMEM((CHV,), jnp.int32),
                       pltpu.VMEM((CH,), jnp.int32),
                       pltpu.VMEM((CHV, 128), F32),
                       pltpu.VMEM_SHARED((HP, 128), F32),
                       pltpu.VMEM_SHARED((DR, 128), F32)],
    )
    def k(vals_hbm, vd_hbm, iq_hbm, id_hbm, zu_hbm, u_hbm, d_hbm,
          idx_v, idxd_v, vv, sh_u, sh_d):
        c = lax.axis_index("c")
        s = lax.axis_index("s")
        for g in range(NQ):
            r0 = s * stripe
            pltpu.sync_copy(zu_hbm.at[pl.ds(r0, stripe)],
                            sh_u.at[pl.ds(r0, stripe)])
            plsc.subcore_barrier()

            def body(i, carry):
                if nchan == 2:
                    be = s * per_sub + i * CHV
                    bv = c * e + be
                else:
                    be = c * half + s * per_sub + i * CHV
                    bv = be
                pltpu.sync_copy(iq_hbm.at[pl.ds(g * e + be, CHV)], idx_v)
                pltpu.sync_copy(vals_hbm.at[pl.ds(bv, CHV)], vv)
                pltpu.sync_copy(vv, sh_u.at[idx_v], add=True)
                return carry

            lax.fori_loop(0, nch_v, body, 0)
            plsc.subcore_barrier()
            pltpu.sync_copy(
                sh_u.at[pl.ds(s * fl, fl)],
                u_hbm.at[pl.ds(c * (NQ * HF) + g * HF + s * fl, fl)])
            plsc.subcore_barrier()
        rd = s * sd
        pltpu.sync_copy(zu_hbm.at[pl.ds(rd, sd)], sh_d.at[pl.ds(rd, sd)])
        plsc.subcore_barrier()

        def body_d(i, carry):
            b = c * half + s * per_sub_d + i * CH
            pltpu.sync_copy(id_hbm.at[pl.ds(b, CH)], idxd_v)
            pltpu.sync_copy(vd_hbm.at[pl.ds(b, CH)], vv.at[pl.ds(0, CH)])
            pltpu.sync_copy(vv.at[pl.ds(0, CH)], sh_d.at[idxd_v], add=True)
            return carry

        lax.fori_loop(0, nch_d, body_d, 0)
        plsc.subcore_barrier()
        pltpu.sync_copy(sh_d.at[pl.ds(rd, sd)],
                        d_hbm.at[pl.ds(c * DR + rd, sd)])

    return k


def _gat_layer(x, src, dst, dst2, e_feat, g, p, big):
    n = x.shape[0]
    e = e_feat.shape[0]
    br_n = 1000
    br_e = 1000
    if big:
        d = p['Wl'].shape[1]
        wl, wr, we, wg, att = p['Wl'], p['Wr'], p['We'], p['Wg'], p['att']
        b = p['b']
    else:
        d = 128

        def pad_w(w):
            return jnp.pad(w, ((0, 0), (0, d - w.shape[1])))

        wl, wr, we, wg = map(pad_w, (p['Wl'], p['Wr'], p['We'], p['Wg']))
        att = jnp.pad(p['att'], (0, d - p['att'].shape[0]))
        b = jnp.pad(p['b'], (0, 16 - p['b'].shape[0]))
    xl = _mm(x, wl, br_n)
    xr = _mm(x, wr, br_n)
    eg = _eg(e_feat, we, g, wg, br_e)
    mi, mj = _make_gather(n, e, d)(xl, xr, dst, src)
    lg, bm = _logits(mi, mj, eg, att, br_e)
    vd, iq, idd = _aux(lg, bm, dst2, br_e)
    iqf = iq.reshape(NQ * e)
    idf = idd.reshape(e)
    zu = jnp.zeros((HP, 128), F32)
    if big:
        vals = _vals_big(lg, bm, mj, br_e).reshape(2 * e, 128)
        u, dd = _make_scatter(n, e, 2)(vals, vd, iqf, idf, zu)
    else:
        vals = _vals_small(lg, bm, mj, br_e)
        u, dd = _make_scatter(n, e, 1)(vals, vd, iqf, idf, zu)
    u = u.reshape(NC, NQ * HF, 128)
    dd = dd.reshape(NC, DR, 128)
    den = (dd[0] + dd[1]).reshape(DR * 128)[:n].reshape(n, 1)
    if big:
        return _comb_big(u, den, b, n, br_n)
    return _comb_small(u[0] + u[1], den, b, n, br_n)


def kernel(nodes_matrix, edge_index, global_vector, edges_matrix, params):
    src = edge_index[0]
    dst = edge_index[1]
    dst2 = dst.reshape(-1, 1)
    g = global_vector
    ef = edges_matrix
    h = _gat_layer(nodes_matrix, src, dst, dst2, ef, g, params['a0'], True)
    h = _gat_layer(h, src, dst, dst2, ef, g, params['a1'], True)
    a = _gat_layer(h, src, dst, dst2, ef, g, params['ah'], False)
    c = _gat_layer(nodes_matrix, src, dst, dst2, ef, g, params['c0'], True)
    c = _gat_layer(c, src, dst, dst2, ef, g, params['c1'], True)
    v = _gat_layer(c, src, dst, dst2, ef, g, params['ch'], False)
    o = _head(a, v)
    flat = jnp.concatenate([o[0, 0:1], o[0, 1:2], a[:, :14].reshape(-1)])
    return (flat, o[0, 2], o[0, 3], o[0, 4])
